# Initial kernel scaffold; baseline (speedup 1.0000x reference)
import jax
import jax.numpy as jnp
from jax.experimental import pallas as pl


def kernel(tokens_idx, seg, positional_embedding, voc_table, seg_table, gamma, beta):
    raise NotImplementedError("write your pallas kernel here")


# speedup vs baseline: 2.4366x; 2.4366x over previous
import functools

import jax
import jax.numpy as jnp
from jax import lax
from jax.experimental import pallas as pl
from jax.experimental.pallas import tpu as pltpu
from jax.experimental.pallas import tpu_sc as plsc

_EPS = 1e-5


@functools.partial(jax.jit, static_argnames=("L",))
def _emb_ln(tok_flat, seg_flat, combo, voc_table, gb, *, L):
    N = tok_flat.shape[0]
    V, D = voc_table.shape
    NSEGL = combo.shape[0]

    info = plsc.get_sparse_core_info()
    NC, NS = info.num_cores, info.num_subcores
    NW = NC * NS
    assert N % NW == 0
    per_w = N // NW
    CHUNK = 2 * L
    assert per_w % CHUNK == 0
    nchunks = per_w // CHUNK
    NVEC = D // 16

    mesh = plsc.VectorSubcoreMesh(core_axis_name="c", subcore_axis_name="s")

    @functools.partial(
        pl.kernel,
        mesh=mesh,
        out_type=jax.ShapeDtypeStruct((N, D), jnp.float32),
        compiler_params=pltpu.CompilerParams(
            needs_layout_passes=False, use_tc_tiling_on_sc=False),
        scratch_types=[
            pltpu.VMEM((CHUNK,), jnp.int32),
            pltpu.VMEM((CHUNK + 16,), jnp.int32),
            pltpu.VMEM((CHUNK, D), jnp.float32),
            pltpu.VMEM((NSEGL, D), jnp.float32),
            pltpu.VMEM((2 * D,), jnp.float32),
            pltpu.SemaphoreType.DMA,
        ],
    )
    def k(tok_hbm, seg_hbm, combo_hbm, voc_hbm, gb_hbm, out_hbm,
          idx_v, seg_v, rows_v, combo_v, gb_v, sem):
        wid = lax.axis_index("s") * NC + lax.axis_index("c")
        pltpu.sync_copy(combo_hbm, combo_v)
        pltpu.sync_copy(gb_hbm, gb_v)
        gvecs = [gb_v[pl.ds(16 * j, 16)] for j in range(NVEC)]
        bvecs = [gb_v[pl.ds(D + 16 * j, 16)] for j in range(NVEC)]
        base0 = wid * per_w

        def chunk_body(g, carry):
            base = base0 + g * CHUNK
            pltpu.sync_copy(tok_hbm.at[pl.ds(base, CHUNK)], idx_v)
            pltpu.sync_copy(seg_hbm.at[pl.ds(base, CHUNK)], seg_v.at[pl.ds(0, CHUNK)])
            pltpu.async_copy(voc_hbm.at[idx_v], rows_v, sem).wait()

            def row_body(r, c2):
                pos = lax.rem(r, L)
                ci = seg_v[pl.ds(r, 16)][0] * L + pos
                x = [rows_v[r, pl.ds(16 * j, 16)] + combo_v[ci, pl.ds(16 * j, 16)]
                     for j in range(NVEC)]
                S = jnp.sum((x[0] + x[1]) + (x[2] + x[3]))
                q = x[0] * x[0]
                for j in range(1, NVEC):
                    q = q + x[j] * x[j]
                Q = jnp.sum(q)
                mean = S * (1.0 / D)
                var = Q * (1.0 / D) - mean * mean
                vv = jnp.full((16,), var + _EPS, jnp.float32)
                mv = jnp.full((16,), mean, jnp.float32)
                bits = lax.bitcast_convert_type(vv, jnp.int32)
                bits = 0x5F3759DF - lax.shift_right_arithmetic(bits, 1)
                y = lax.bitcast_convert_type(bits, jnp.float32)
                h = vv * 0.5
                for _ in range(3):
                    y = y * (1.5 - h * y * y)
                for j in range(NVEC):
                    scale = y * gvecs[j]
                    rows_v[r, pl.ds(16 * j, 16)] = (x[j] - mv) * scale + bvecs[j]
                return c2

            lax.fori_loop(0, CHUNK, row_body, 0)
            pltpu.sync_copy(rows_v, out_hbm.at[pl.ds(base, CHUNK)])
            return carry

        lax.fori_loop(0, nchunks, chunk_body, 0)

    return k(tok_flat, seg_flat, combo, voc_table, gb)


def kernel(tokens_idx, seg, positional_embedding, voc_table, seg_table, gamma, beta):
    B, L = tokens_idx.shape
    D = voc_table.shape[1]
    combo = (seg_table[:, None, :] + positional_embedding[None, :, :]).reshape(-1, D)
    gb = jnp.concatenate([gamma, beta])
    out = _emb_ln(tokens_idx.reshape(-1), seg.reshape(-1), combo, voc_table, gb, L=L)
    return out.reshape(B, L, D)


# speedup vs baseline: 3.1862x; 1.3076x over previous
import functools

import jax
import jax.numpy as jnp
from jax import lax
from jax.experimental import pallas as pl
from jax.experimental.pallas import tpu as pltpu
from jax.experimental.pallas import tpu_sc as plsc

_EPS = 1e-5


@functools.partial(jax.jit, static_argnames=("L",))
def _emb_ln(tok_flat, seg_flat, combo, voc_table, gb, *, L):
    N = tok_flat.shape[0]
    V, D = voc_table.shape
    NSEGL = combo.shape[0]

    info = plsc.get_sparse_core_info()
    NC, NS = info.num_cores, info.num_subcores
    NW = NC * NS
    assert N % NW == 0
    per_w = N // NW
    CHUNK = 2 * L
    assert per_w % CHUNK == 0
    nchunks = per_w // CHUNK
    NVEC = D // 16
    RB = 16
    assert CHUNK % RB == 0

    mesh = plsc.VectorSubcoreMesh(core_axis_name="c", subcore_axis_name="s")

    @functools.partial(
        pl.kernel,
        mesh=mesh,
        out_type=jax.ShapeDtypeStruct((N, D), jnp.float32),
        compiler_params=pltpu.CompilerParams(
            needs_layout_passes=False, use_tc_tiling_on_sc=False),
        scratch_types=[
            pltpu.VMEM((2, CHUNK), jnp.int32),
            pltpu.VMEM((2, CHUNK), jnp.int32),
            pltpu.VMEM((2, CHUNK, D), jnp.float32),
            pltpu.VMEM((NSEGL, D), jnp.float32),
            pltpu.VMEM((2 * D,), jnp.float32),
            pltpu.SemaphoreType.DMA,
            pltpu.SemaphoreType.DMA,
        ],
    )
    def k(tok_hbm, seg_hbm, combo_hbm, voc_hbm, gb_hbm, out_hbm,
          idx_v, seg_v, rows_v, combo_v, gb_v, gsem0, gsem1):
        wid = lax.axis_index("s") * NC + lax.axis_index("c")
        pltpu.sync_copy(combo_hbm, combo_v)
        pltpu.sync_copy(gb_hbm, gb_v)
        gvecs = [gb_v[pl.ds(16 * j, 16)] for j in range(NVEC)]
        bvecs = [gb_v[pl.ds(D + 16 * j, 16)] for j in range(NVEC)]
        gsems = (gsem0, gsem1)
        base0 = wid * per_w

        def start_gather(g, b):
            base = base0 + g * CHUNK
            pltpu.sync_copy(tok_hbm.at[pl.ds(base, CHUNK)], idx_v.at[b])
            pltpu.sync_copy(seg_hbm.at[pl.ds(base, CHUNK)], seg_v.at[b])
            pltpu.async_copy(voc_hbm.at[idx_v.at[b]], rows_v.at[b], gsems[b])

        def wait_gather(b):
            pltpu.make_async_copy(
                voc_hbm.at[idx_v.at[b]], rows_v.at[b], gsems[b]).wait()

        def do_chunk(g, b):
            wait_gather(b)

            @pl.when(g + 1 < nchunks)
            def _():
                start_gather(g + 1, 1 - b)

            def blk_body(m, carry):
                R = m * RB
                sv = seg_v[b, pl.ds(R, RB)]
                for kk in range(RB):
                    r = R + kk
                    pos = lax.rem(r, L)
                    ci = sv[kk] * L + pos
                    x = [rows_v[b, r, pl.ds(16 * j, 16)]
                         + combo_v[ci, pl.ds(16 * j, 16)]
                         for j in range(NVEC)]
                    S = jnp.sum((x[0] + x[1]) + (x[2] + x[3]))
                    q = x[0] * x[0]
                    for j in range(1, NVEC):
                        q = q + x[j] * x[j]
                    Q = jnp.sum(q)
                    mean = S * (1.0 / D)
                    var = Q * (1.0 / D) - mean * mean
                    vv = jnp.full((16,), var + _EPS, jnp.float32)
                    mv = jnp.full((16,), mean, jnp.float32)
                    bits = lax.bitcast_convert_type(vv, jnp.int32)
                    bits = 0x5F3759DF - lax.shift_right_arithmetic(bits, 1)
                    y = lax.bitcast_convert_type(bits, jnp.float32)
                    h = vv * 0.5
                    for _ in range(2):
                        y = y * (1.5 - h * y * y)
                    for j in range(NVEC):
                        scale = y * gvecs[j]
                        rows_v[b, r, pl.ds(16 * j, 16)] = (
                            (x[j] - mv) * scale + bvecs[j])
                return carry

            lax.fori_loop(0, CHUNK // RB, blk_body, 0)
            base = base0 + g * CHUNK
            pltpu.sync_copy(rows_v.at[b], out_hbm.at[pl.ds(base, CHUNK)])

        start_gather(0, 0)

        def pair_body(p, carry):
            do_chunk(2 * p, 0)
            do_chunk(2 * p + 1, 1)
            return carry

        lax.fori_loop(0, nchunks // 2, pair_body, 0)

    return k(tok_flat, seg_flat, combo, voc_table, gb)


def kernel(tokens_idx, seg, positional_embedding, voc_table, seg_table, gamma, beta):
    B, L = tokens_idx.shape
    D = voc_table.shape[1]
    combo = (seg_table[:, None, :] + positional_embedding[None, :, :]).reshape(-1, D)
    gb = jnp.concatenate([gamma, beta])
    out = _emb_ln(tokens_idx.reshape(-1), seg.reshape(-1), combo, voc_table, gb, L=L)
    return out.reshape(B, L, D)
